# Initial kernel scaffold; baseline (speedup 1.0000x reference)
import jax
import jax.numpy as jnp
from jax.experimental import pallas as pl


def kernel(word, age, word_table, age_table):
    raise NotImplementedError("write your pallas kernel here")


# speedup vs baseline: 1.7609x; 1.7609x over previous
import functools

import jax
import jax.numpy as jnp
from jax import lax
from jax.experimental import pallas as pl
from jax.experimental.pallas import tpu as pltpu
from jax.experimental.pallas import tpu_sc as plsc

BATCH = 4096
HIST = 200
WORD_DIM = 64
AGE_DIM = 32
OUT_DIM = WORD_DIM + AGE_DIM

NTOK = BATCH * HIST
IDXW = 128
NROWS = NTOK // IDXW
NWORKERS = 32
ROWS_PER_W = NROWS // NWORKERS
RPC = 4
CHUNK = RPC * IDXW
NCHUNKS = ROWS_PER_W // RPC


def _body(widx_hbm, aidx_hbm, wtab_hbm, atab_hbm, out_hbm,
          widx_v, aidx_v, wrows_v, arows_v, sem):
    cid = lax.axis_index("c")
    sid = lax.axis_index("s")
    wid = sid * 2 + cid

    def chunk(i, carry):
        row0 = wid * ROWS_PER_W + i * RPC
        pltpu.sync_copy(widx_hbm.at[pl.ds(row0, RPC)], widx_v)
        pltpu.sync_copy(aidx_hbm.at[pl.ds(row0, RPC)], aidx_v)
        copies = []
        for j in range(RPC):
            copies.append(pltpu.async_copy(
                wtab_hbm.at[widx_v.at[j]],
                wrows_v.at[pl.ds(j * IDXW, IDXW)], sem))
            copies.append(pltpu.async_copy(
                atab_hbm.at[aidx_v.at[j]],
                arows_v.at[pl.ds(j * IDXW, IDXW)], sem))
        for c in copies:
            c.wait()
        base = row0 * IDXW
        pltpu.sync_copy(wrows_v, out_hbm.at[pl.ds(base, CHUNK), pl.ds(0, WORD_DIM)])
        pltpu.sync_copy(arows_v, out_hbm.at[pl.ds(base, CHUNK), pl.ds(WORD_DIM, AGE_DIM)])
        return carry

    lax.fori_loop(0, NCHUNKS, chunk, 0)


@jax.jit
def _embed(widx, aidx, word_table, age_table):
    kern = pl.kernel(
        _body,
        out_type=jax.ShapeDtypeStruct((NTOK, OUT_DIM), jnp.float32),
        mesh=plsc.VectorSubcoreMesh(core_axis_name="c", subcore_axis_name="s"),
        scratch_types=[
            pltpu.VMEM((RPC, IDXW), jnp.int32),
            pltpu.VMEM((RPC, IDXW), jnp.int32),
            pltpu.VMEM((CHUNK, WORD_DIM), jnp.float32),
            pltpu.VMEM((CHUNK, AGE_DIM), jnp.float32),
            pltpu.SemaphoreType.DMA,
        ],
        compiler_params=pltpu.CompilerParams(use_tc_tiling_on_sc=False),
    )
    return kern(widx, aidx, word_table, age_table)


def kernel(word, age, word_table, age_table):
    widx = word.astype(jnp.int32).reshape(NROWS, IDXW)
    aidx = age.astype(jnp.int32).reshape(NROWS, IDXW)
    out = _embed(widx, aidx, word_table, age_table)
    return out.reshape(BATCH, HIST, OUT_DIM)
